# Initial kernel scaffold; baseline (speedup 1.0000x reference)
import jax
import jax.numpy as jnp
from jax.experimental import pallas as pl


def kernel(x_product, x_demographic, x_platform, edge_index_targets, edge_index_rev_targets, edge_index_uses, edge_index_rev_uses, edge_index_self, Wl_targets, bl_targets, Wr_targets, Wl_rev_targets, bl_rev_targets, Wr_rev_targets, Wl_uses, bl_uses, Wr_uses, Wl_rev_uses, bl_rev_uses, Wr_rev_uses, Wl_self, bl_self, Wr_self, lin_W, lin_b):
    raise NotImplementedError("write your pallas kernel here")


# speedup vs baseline: 7.9853x; 7.9853x over previous
import functools

import jax
import jax.numpy as jnp
from jax import lax
from jax.experimental import pallas as pl
from jax.experimental.pallas import tpu as pltpu
from jax.experimental.pallas import tpu_sc as plsc

D_IN = 32
CHUNK = 128
N_SUBCORES = 16
ROW_STRIDE = 3200
ROW_STEP = 200


def _sc_accumulate(n_dst, n_chunks, src_a, dst_a, src_b, dst_b, x_a, x_b):
    trips = (n_chunks + N_SUBCORES - 1) // N_SUBCORES

    mesh = plsc.VectorSubcoreMesh(core_axis_name="c", subcore_axis_name="s")

    @functools.partial(
        pl.kernel,
        out_type=(
            jax.ShapeDtypeStruct((n_dst, D_IN), jnp.float32),
            jax.ShapeDtypeStruct((n_dst,), jnp.float32),
            jax.ShapeDtypeStruct((n_dst, D_IN), jnp.float32),
            jax.ShapeDtypeStruct((n_dst,), jnp.float32),
        ),
        mesh=mesh,
        scratch_types=(
            pltpu.VMEM((CHUNK,), jnp.int32),
            pltpu.VMEM((CHUNK,), jnp.int32),
            pltpu.VMEM((CHUNK, D_IN), jnp.float32),
            pltpu.VMEM((CHUNK,), jnp.float32),
            pltpu.VMEM((ROW_STEP, D_IN), jnp.float32),
            pltpu.VMEM((ROW_STEP,), jnp.float32),
            pltpu.VMEM_SHARED((n_dst, D_IN), jnp.float32),
            pltpu.VMEM_SHARED((n_dst,), jnp.float32),
            pltpu.SemaphoreType.DMA,
        ),
        compiler_params=pltpu.CompilerParams(use_tc_tiling_on_sc=False),
    )
    def k(src_a_h, dst_a_h, src_b_h, dst_b_h, x_a_h, x_b_h,
          acc_a_h, cnt_a_h, acc_b_h, cnt_b_h,
          src_v, dst_v, rows_v, ones_v, zrow_v, zcnt_v, acc_sh, cnt_sh, sem):
        core = lax.axis_index("c")
        tile = lax.axis_index("s")

        zero16 = jnp.zeros((16,), jnp.float32)
        one16 = jnp.ones((16,), jnp.float32)

        def init_ones(i, _):
            ones_v[pl.ds(i * 16, 16)] = one16
            return 0
        lax.fori_loop(0, CHUNK // 16, init_ones, 0)

        def init_zcnt(i, _):
            zcnt_v[pl.ds(i * 16, 16)] = zero16
            return 0
        lax.fori_loop(0, ROW_STEP // 16, init_zcnt, 0)

        def init_zrow(r, _):
            zrow_v[r, pl.ds(0, 16)] = zero16
            zrow_v[r, pl.ds(16, 16)] = zero16
            return 0
        lax.fori_loop(0, ROW_STEP, init_zrow, 0)

        for j in range(ROW_STRIDE // ROW_STEP):
            off = tile * ROW_STRIDE + j * ROW_STEP

            @pl.when(off < n_dst)
            def _():
                pltpu.sync_copy(zrow_v, acc_sh.at[pl.ds(off, ROW_STEP)])
                pltpu.sync_copy(zcnt_v, cnt_sh.at[pl.ds(off, ROW_STEP)])
        plsc.subcore_barrier()

        def body(j, _):
            c = tile + N_SUBCORES * j

            @pl.when(c < n_chunks)
            def _():
                @pl.when(core == 0)
                def _():
                    pltpu.sync_copy(src_a_h.at[pl.ds(c * CHUNK, CHUNK)], src_v)
                    pltpu.sync_copy(dst_a_h.at[pl.ds(c * CHUNK, CHUNK)], dst_v)
                    pltpu.async_copy(x_a_h.at[src_v], rows_v, sem).wait()

                @pl.when(core == 1)
                def _():
                    pltpu.sync_copy(src_b_h.at[pl.ds(c * CHUNK, CHUNK)], src_v)
                    pltpu.sync_copy(dst_b_h.at[pl.ds(c * CHUNK, CHUNK)], dst_v)
                    pltpu.async_copy(x_b_h.at[src_v], rows_v, sem).wait()

                pltpu.sync_copy(rows_v, acc_sh.at[dst_v], add=True)
                pltpu.sync_copy(ones_v, cnt_sh.at[dst_v], add=True)
            return 0
        lax.fori_loop(0, trips, body, 0)
        plsc.subcore_barrier()

        for j in range(ROW_STRIDE // ROW_STEP):
            off = tile * ROW_STRIDE + j * ROW_STEP

            @pl.when(off < n_dst)
            def _():
                @pl.when(core == 0)
                def _():
                    pltpu.sync_copy(acc_sh.at[pl.ds(off, ROW_STEP)],
                                    acc_a_h.at[pl.ds(off, ROW_STEP)])
                    pltpu.sync_copy(cnt_sh.at[pl.ds(off, ROW_STEP)],
                                    cnt_a_h.at[pl.ds(off, ROW_STEP)])

                @pl.when(core == 1)
                def _():
                    pltpu.sync_copy(acc_sh.at[pl.ds(off, ROW_STEP)],
                                    acc_b_h.at[pl.ds(off, ROW_STEP)])
                    pltpu.sync_copy(cnt_sh.at[pl.ds(off, ROW_STEP)],
                                    cnt_b_h.at[pl.ds(off, ROW_STEP)])

    return k(src_a, dst_a, src_b, dst_b, x_a, x_b)


def _tc_body(acc_a, cnt_a, acc_b, cnt_b, xp,
             wl_a, bl_a, wr_a, wl_b, bl_b, wr_b, lin_w, lin_b, out):
    mean_a = acc_a[...] / jnp.maximum(cnt_a[...], 1.0)
    mean_b = acc_b[...] / jnp.maximum(cnt_b[...], 1.0)
    h = (jnp.dot(mean_a, wl_a[...], preferred_element_type=jnp.float32)
         + jnp.dot(mean_b, wl_b[...], preferred_element_type=jnp.float32)
         + jnp.dot(xp[...], wr_a[...] + wr_b[...],
                   preferred_element_type=jnp.float32)
         + bl_a[...] + bl_b[...]) * 0.5
    h = jnp.where(h >= 0, h, 0.01 * h)
    out[...] = (jnp.dot(h, lin_w[...], preferred_element_type=jnp.float32)
                + lin_b[...])


def _tc_epilogue(acc_a, cnt_a, acc_b, cnt_b, x_product,
                 wl_a, bl_a, wr_a, wl_b, bl_b, wr_b, lin_w, lin_b):
    n = x_product.shape[0]
    n_cls = lin_w.shape[1]
    bm = 1000
    grid = (n // bm,)
    row_spec = lambda w: pl.BlockSpec((bm, w), lambda i: (i, 0))
    full = lambda a: pl.BlockSpec(a.shape, lambda i: (0,) * a.ndim)
    return pl.pallas_call(
        _tc_body,
        grid=grid,
        in_specs=[
            row_spec(D_IN), row_spec(1), row_spec(D_IN), row_spec(1),
            row_spec(D_IN),
            full(wl_a), full(bl_a), full(wr_a),
            full(wl_b), full(bl_b), full(wr_b),
            full(lin_w), full(lin_b),
        ],
        out_specs=row_spec(n_cls),
        out_shape=jax.ShapeDtypeStruct((n, n_cls), jnp.float32),
    )(acc_a, cnt_a, acc_b, cnt_b, x_product,
      wl_a, bl_a, wr_a, wl_b, bl_b, wr_b, lin_w, lin_b)


def kernel(x_product, x_demographic, x_platform, edge_index_targets,
           edge_index_rev_targets, edge_index_uses, edge_index_rev_uses,
           edge_index_self,
           Wl_targets, bl_targets, Wr_targets,
           Wl_rev_targets, bl_rev_targets, Wr_rev_targets,
           Wl_uses, bl_uses, Wr_uses,
           Wl_rev_uses, bl_rev_uses, Wr_rev_uses,
           Wl_self, bl_self, Wr_self,
           lin_W, lin_b):
    e = edge_index_rev_targets.shape[1]
    n_chunks = e // CHUNK
    n_prod = x_product.shape[0]

    acc_a, cnt_a, acc_b, cnt_b = _sc_accumulate(
        n_prod, n_chunks,
        edge_index_rev_targets[0], edge_index_rev_targets[1],
        edge_index_self[0], edge_index_self[1],
        x_demographic, x_product)

    return _tc_epilogue(
        acc_a, cnt_a.reshape(n_prod, 1), acc_b, cnt_b.reshape(n_prod, 1),
        x_product,
        Wl_rev_targets, bl_rev_targets.reshape(1, -1), Wr_rev_targets,
        Wl_self, bl_self.reshape(1, -1), Wr_self,
        lin_W, lin_b.reshape(1, -1))


# speedup vs baseline: 11.2772x; 1.4122x over previous
import functools

import jax
import jax.numpy as jnp
from jax import lax
from jax.experimental import pallas as pl
from jax.experimental.pallas import tpu as pltpu
from jax.experimental.pallas import tpu_sc as plsc

D_IN = 32
LANES = 128
SUBS = 8
NB = 50
N_SUBCORES = 16
EPT = NB * SUBS * LANES
SPROWS = 51200
ROW_STRIDE = 3200
ROW_STEP = 200


def _prep_edges(edge_index, src_offset, dump_idx):
    src = edge_index[0].reshape(N_SUBCORES, -1)
    dst = edge_index[1].reshape(N_SUBCORES, -1)
    pad = EPT - src.shape[1]
    src = jnp.pad(src, ((0, 0), (0, pad))) + src_offset
    dst = jnp.pad(dst, ((0, 0), (0, pad)), constant_values=dump_idx)
    src = src.reshape(N_SUBCORES, NB, SUBS, LANES)
    dst = dst.reshape(N_SUBCORES, NB, SUBS, LANES)
    return jnp.stack([src, dst], axis=2)


def _sc_accumulate(n_dst, edges_all, x_all):
    mesh = plsc.VectorSubcoreMesh(core_axis_name="c", subcore_axis_name="s")

    @functools.partial(
        pl.kernel,
        out_type=(
            jax.ShapeDtypeStruct((n_dst, D_IN), jnp.float32),
            jax.ShapeDtypeStruct((n_dst,), jnp.float32),
            jax.ShapeDtypeStruct((n_dst, D_IN), jnp.float32),
            jax.ShapeDtypeStruct((n_dst,), jnp.float32),
        ),
        mesh=mesh,
        scratch_types=(
            pltpu.VMEM((2, SUBS, LANES), jnp.int32),
            pltpu.VMEM((2, SUBS, LANES), jnp.int32),
            pltpu.VMEM((LANES, D_IN), jnp.float32),
            pltpu.VMEM((LANES, D_IN), jnp.float32),
            pltpu.VMEM((LANES,), jnp.float32),
            pltpu.VMEM((ROW_STEP, D_IN), jnp.float32),
            pltpu.VMEM((ROW_STEP,), jnp.float32),
            pltpu.VMEM_SHARED((SPROWS, D_IN), jnp.float32),
            pltpu.VMEM_SHARED((SPROWS,), jnp.float32),
            pltpu.SemaphoreType.DMA,
            pltpu.SemaphoreType.DMA,
            pltpu.SemaphoreType.DMA,
            pltpu.SemaphoreType.DMA,
        ),
        compiler_params=pltpu.CompilerParams(use_tc_tiling_on_sc=False),
    )
    def k(edges_h, x_h, acc_a_h, cnt_a_h, acc_b_h, cnt_b_h,
          ib0, ib1, rows0, rows1, ones_v, zrow_v, zcnt_v, acc_sh, cnt_sh,
          si0, si1, sg0, sg1):
        core = lax.axis_index("c")
        tile = lax.axis_index("s")
        ib = (ib0, ib1)
        si = (si0, si1)
        rows = (rows0, rows1)
        sg = (sg0, sg1)

        zero16 = jnp.zeros((16,), jnp.float32)
        one16 = jnp.ones((16,), jnp.float32)

        def init_ones(i, _):
            ones_v[pl.ds(i * 16, 16)] = one16
            return 0
        lax.fori_loop(0, LANES // 16, init_ones, 0)

        def init_zcnt(i, _):
            zcnt_v[pl.ds(i * 16, 16)] = zero16
            return 0
        lax.fori_loop(0, ROW_STEP // 16, init_zcnt, 0)

        def init_zrow(r, _):
            zrow_v[r, pl.ds(0, 16)] = zero16
            zrow_v[r, pl.ds(16, 16)] = zero16
            return 0
        lax.fori_loop(0, ROW_STEP, init_zrow, 0)

        for j in range(ROW_STRIDE // ROW_STEP):
            off = tile * ROW_STRIDE + j * ROW_STEP
            pltpu.sync_copy(zrow_v, acc_sh.at[pl.ds(off, ROW_STEP)])
            pltpu.sync_copy(zcnt_v, cnt_sh.at[pl.ds(off, ROW_STEP)])
        plsc.subcore_barrier()

        def gwait(r):
            pltpu.make_async_copy(x_h.at[pl.ds(0, LANES)], rows[r],
                                  sg[r]).wait()

        pltpu.sync_copy(edges_h.at[core, tile, 0], ib[0])
        pltpu.async_copy(x_h.at[ib[0].at[0, 0]], rows[0], sg[0])

        def pair_body(m, _):
            for b in (0, 1):
                i = 2 * m + b
                inxt = jnp.minimum(i + 1, NB - 1)
                pltpu.async_copy(edges_h.at[core, tile, inxt], ib[1 - b],
                                 si[1 - b])
                rk = 0
                for s in range(SUBS):
                    if s < SUBS - 1:
                        pltpu.async_copy(x_h.at[ib[b].at[0, s + 1]],
                                         rows[1 - rk], sg[1 - rk])
                    else:
                        pltpu.make_async_copy(edges_h.at[core, tile, inxt],
                                              ib[1 - b], si[1 - b]).wait()
                        pltpu.async_copy(x_h.at[ib[1 - b].at[0, 0]],
                                         rows[1 - rk], sg[1 - rk])
                    gwait(rk)
                    pltpu.sync_copy(rows[rk], acc_sh.at[ib[b].at[1, s]],
                                    add=True)
                    pltpu.sync_copy(ones_v, cnt_sh.at[ib[b].at[1, s]],
                                    add=True)
                    rk = 1 - rk
            return 0
        lax.fori_loop(0, NB // 2, pair_body, 0)
        gwait(0)
        plsc.subcore_barrier()

        for j in range(ROW_STRIDE // ROW_STEP):
            off = tile * ROW_STRIDE + j * ROW_STEP

            @pl.when(off < n_dst)
            def _():
                @pl.when(core == 0)
                def _():
                    pltpu.sync_copy(acc_sh.at[pl.ds(off, ROW_STEP)],
                                    acc_a_h.at[pl.ds(off, ROW_STEP)])
                    pltpu.sync_copy(cnt_sh.at[pl.ds(off, ROW_STEP)],
                                    cnt_a_h.at[pl.ds(off, ROW_STEP)])

                @pl.when(core == 1)
                def _():
                    pltpu.sync_copy(acc_sh.at[pl.ds(off, ROW_STEP)],
                                    acc_b_h.at[pl.ds(off, ROW_STEP)])
                    pltpu.sync_copy(cnt_sh.at[pl.ds(off, ROW_STEP)],
                                    cnt_b_h.at[pl.ds(off, ROW_STEP)])

    return k(edges_all, x_all)


def _tc_body(acc_a, cnt_a, acc_b, cnt_b, xp,
             wl_a, bl_a, wr_a, wl_b, bl_b, wr_b, lin_w, lin_b, out):
    mean_a = acc_a[...] / jnp.maximum(cnt_a[...], 1.0)
    mean_b = acc_b[...] / jnp.maximum(cnt_b[...], 1.0)
    h = (jnp.dot(mean_a, wl_a[...], preferred_element_type=jnp.float32)
         + jnp.dot(mean_b, wl_b[...], preferred_element_type=jnp.float32)
         + jnp.dot(xp[...], wr_a[...] + wr_b[...],
                   preferred_element_type=jnp.float32)
         + bl_a[...] + bl_b[...]) * 0.5
    h = jnp.where(h >= 0, h, 0.01 * h)
    out[...] = (jnp.dot(h, lin_w[...], preferred_element_type=jnp.float32)
                + lin_b[...])


def _tc_epilogue(acc_a, cnt_a, acc_b, cnt_b, x_product,
                 wl_a, bl_a, wr_a, wl_b, bl_b, wr_b, lin_w, lin_b):
    n = x_product.shape[0]
    n_cls = lin_w.shape[1]
    bm = 1000
    grid = (n // bm,)
    row_spec = lambda w: pl.BlockSpec((bm, w), lambda i: (i, 0))
    full = lambda a: pl.BlockSpec(a.shape, lambda i: (0,) * a.ndim)
    return pl.pallas_call(
        _tc_body,
        grid=grid,
        in_specs=[
            row_spec(D_IN), row_spec(1), row_spec(D_IN), row_spec(1),
            row_spec(D_IN),
            full(wl_a), full(bl_a), full(wr_a),
            full(wl_b), full(bl_b), full(wr_b),
            full(lin_w), full(lin_b),
        ],
        out_specs=row_spec(n_cls),
        out_shape=jax.ShapeDtypeStruct((n, n_cls), jnp.float32),
    )(acc_a, cnt_a, acc_b, cnt_b, x_product,
      wl_a, bl_a, wr_a, wl_b, bl_b, wr_b, lin_w, lin_b)


def kernel(x_product, x_demographic, x_platform, edge_index_targets,
           edge_index_rev_targets, edge_index_uses, edge_index_rev_uses,
           edge_index_self,
           Wl_targets, bl_targets, Wr_targets,
           Wl_rev_targets, bl_rev_targets, Wr_rev_targets,
           Wl_uses, bl_uses, Wr_uses,
           Wl_rev_uses, bl_rev_uses, Wr_rev_uses,
           Wl_self, bl_self, Wr_self,
           lin_W, lin_b):
    n_prod = x_product.shape[0]
    n_demo = x_demographic.shape[0]

    x_all = jnp.concatenate([x_demographic, x_product], axis=0)
    edges_all = jnp.stack([
        _prep_edges(edge_index_rev_targets, 0, n_prod),
        _prep_edges(edge_index_self, n_demo, n_prod),
    ], axis=0)

    acc_a, cnt_a, acc_b, cnt_b = _sc_accumulate(n_prod, edges_all, x_all)

    return _tc_epilogue(
        acc_a, cnt_a.reshape(n_prod, 1), acc_b, cnt_b.reshape(n_prod, 1),
        x_product,
        Wl_rev_targets, bl_rev_targets.reshape(1, -1), Wr_rev_targets,
        Wl_self, bl_self.reshape(1, -1), Wr_self,
        lin_W, lin_b.reshape(1, -1))


# speedup vs baseline: 13.1341x; 1.1647x over previous
import functools

import jax
import jax.numpy as jnp
from jax import lax
from jax.experimental import pallas as pl
from jax.experimental.pallas import tpu as pltpu
from jax.experimental.pallas import tpu_sc as plsc

D_IN = 32
LANES = 128
SUBS = 8
NB = 50
N_SUBCORES = 16
EPT = NB * SUBS * LANES
SPROWS = 50176
ROW_STRIDE = 3200
ROW_STEP = 200


def _prep_edges(edge_index, src_offset, dump_idx):
    src = edge_index[0].reshape(N_SUBCORES, -1)
    dst = edge_index[1].reshape(N_SUBCORES, -1)
    pad = EPT - src.shape[1]
    src = jnp.pad(src, ((0, 0), (0, pad))) + src_offset
    dst = jnp.pad(dst, ((0, 0), (0, pad)), constant_values=dump_idx)
    src = src.reshape(N_SUBCORES, NB, SUBS, LANES)
    dst = dst.reshape(N_SUBCORES, NB, SUBS, LANES)
    return jnp.stack([src, dst], axis=2)


def _sc_accumulate(n_dst, edges_all, x_all):
    mesh = plsc.VectorSubcoreMesh(core_axis_name="c", subcore_axis_name="s")

    @functools.partial(
        pl.kernel,
        out_type=(
            jax.ShapeDtypeStruct((n_dst, D_IN), jnp.float32),
            jax.ShapeDtypeStruct((n_dst,), jnp.float32),
            jax.ShapeDtypeStruct((n_dst, D_IN), jnp.float32),
            jax.ShapeDtypeStruct((n_dst,), jnp.float32),
        ),
        mesh=mesh,
        scratch_types=(
            pltpu.VMEM((2, SUBS, LANES), jnp.int32),
            pltpu.VMEM((2, SUBS, LANES), jnp.int32),
            pltpu.VMEM((LANES, D_IN), jnp.float32),
            pltpu.VMEM((LANES, D_IN), jnp.float32),
            pltpu.VMEM((LANES, D_IN), jnp.float32),
            pltpu.VMEM((LANES, D_IN), jnp.float32),
            pltpu.VMEM((LANES,), jnp.float32),
            pltpu.VMEM((ROW_STEP, D_IN), jnp.float32),
            pltpu.VMEM((ROW_STEP,), jnp.float32),
            pltpu.VMEM_SHARED((SPROWS, D_IN), jnp.float32),
            pltpu.VMEM_SHARED((SPROWS,), jnp.float32),
            pltpu.SemaphoreType.DMA,
            pltpu.SemaphoreType.DMA,
            (pltpu.SemaphoreType.DMA,) * 4,
            (pltpu.SemaphoreType.DMA,) * 4,
            (pltpu.SemaphoreType.DMA,) * 4,
        ),
        compiler_params=pltpu.CompilerParams(use_tc_tiling_on_sc=False),
    )
    def k(edges_h, x_h, acc_a_h, cnt_a_h, acc_b_h, cnt_b_h,
          ib0, ib1, rows0, rows1, rows2, rows3, ones_v, zrow_v, zcnt_v,
          acc_sh, cnt_sh, si0, si1, sg, ssc, scnt):
        core = lax.axis_index("c")
        tile = lax.axis_index("s")
        ib = (ib0, ib1)
        si = (si0, si1)
        rows = (rows0, rows1, rows2, rows3)

        zero16 = jnp.zeros((16,), jnp.float32)
        one16 = jnp.ones((16,), jnp.float32)

        def init_ones(i, _):
            ones_v[pl.ds(i * 16, 16)] = one16
            return 0
        lax.fori_loop(0, LANES // 16, init_ones, 0)

        def init_zcnt(i, _):
            zcnt_v[pl.ds(i * 16, 16)] = zero16
            return 0
        lax.fori_loop(0, ROW_STEP // 16, init_zcnt, 0)

        def init_zrow(r, _):
            zrow_v[r, pl.ds(0, 16)] = zero16
            zrow_v[r, pl.ds(16, 16)] = zero16
            return 0
        lax.fori_loop(0, ROW_STEP, init_zrow, 0)

        for j in range(ROW_STRIDE // ROW_STEP):
            off = tile * ROW_STRIDE + j * ROW_STEP

            @pl.when(off < n_dst)
            def _():
                pltpu.sync_copy(zrow_v, acc_sh.at[pl.ds(off, ROW_STEP)])
                pltpu.sync_copy(zcnt_v, cnt_sh.at[pl.ds(off, ROW_STEP)])
        plsc.subcore_barrier()

        def gwait(r):
            pltpu.make_async_copy(x_h.at[pl.ds(0, LANES)], rows[r],
                                  sg[r]).wait()

        def drain_sc(r):
            pltpu.make_async_copy(x_h.at[pl.ds(0, LANES)], rows[r],
                                  ssc[r]).wait()

        def drain_cnt(r):
            pltpu.make_async_copy(cnt_a_h.at[pl.ds(0, LANES)], ones_v,
                                  scnt[r]).wait()

        def emit_batch(i, bb, first=False, last=False):
            inxt = jnp.minimum(i + 1, NB - 1)
            for s in range(SUBS):
                b2 = (s + 2) % 4
                if s < SUBS - 2:
                    if not (first and s < 2):
                        drain_sc(b2)
                        drain_cnt(b2)
                    pltpu.async_copy(x_h.at[ib[bb].at[0, s + 2]],
                                     rows[b2], sg[b2])
                elif not last:
                    if s == SUBS - 2:
                        pltpu.make_async_copy(edges_h.at[core, tile, inxt],
                                              ib[1 - bb], si[1 - bb]).wait()
                    drain_sc(b2)
                    drain_cnt(b2)
                    pltpu.async_copy(x_h.at[ib[1 - bb].at[0, s - (SUBS - 2)]],
                                     rows[b2], sg[b2])
                bc = s % 4
                gwait(bc)
                pltpu.async_copy(rows[bc], acc_sh.at[ib[bb].at[1, s]],
                                 ssc[bc], add=True)
                pltpu.async_copy(ones_v, cnt_sh.at[ib[bb].at[1, s]],
                                 scnt[bc], add=True)
                if s == 1 and not last:
                    pltpu.async_copy(edges_h.at[core, tile, inxt], ib[1 - bb],
                                     si[1 - bb])

        pltpu.sync_copy(edges_h.at[core, tile, 0], ib[0])
        pltpu.async_copy(x_h.at[ib[0].at[0, 0]], rows[0], sg[0])
        pltpu.async_copy(x_h.at[ib[0].at[0, 1]], rows[1], sg[1])

        emit_batch(0, 0, first=True)

        def pair_body(m, _):
            emit_batch(1 + 2 * m, 1)
            emit_batch(2 + 2 * m, 0)
            return 0
        lax.fori_loop(0, (NB - 2) // 2, pair_body, 0)
        emit_batch(NB - 1, 1, last=True)
        for r in range(4):
            drain_sc(r)
            drain_cnt(r)
        plsc.subcore_barrier()

        for j in range(ROW_STRIDE // ROW_STEP):
            off = tile * ROW_STRIDE + j * ROW_STEP

            @pl.when(off < n_dst)
            def _():
                @pl.when(core == 0)
                def _():
                    pltpu.sync_copy(acc_sh.at[pl.ds(off, ROW_STEP)],
                                    acc_a_h.at[pl.ds(off, ROW_STEP)])
                    pltpu.sync_copy(cnt_sh.at[pl.ds(off, ROW_STEP)],
                                    cnt_a_h.at[pl.ds(off, ROW_STEP)])

                @pl.when(core == 1)
                def _():
                    pltpu.sync_copy(acc_sh.at[pl.ds(off, ROW_STEP)],
                                    acc_b_h.at[pl.ds(off, ROW_STEP)])
                    pltpu.sync_copy(cnt_sh.at[pl.ds(off, ROW_STEP)],
                                    cnt_b_h.at[pl.ds(off, ROW_STEP)])

    return k(edges_all, x_all)


def _tc_body(acc_a, cnt_a, acc_b, cnt_b, xp,
             wl_a, bl_a, wr_a, wl_b, bl_b, wr_b, lin_w, lin_b, out):
    mean_a = acc_a[...] / jnp.maximum(cnt_a[...], 1.0)
    mean_b = acc_b[...] / jnp.maximum(cnt_b[...], 1.0)
    h = (jnp.dot(mean_a, wl_a[...], preferred_element_type=jnp.float32)
         + jnp.dot(mean_b, wl_b[...], preferred_element_type=jnp.float32)
         + jnp.dot(xp[...], wr_a[...] + wr_b[...],
                   preferred_element_type=jnp.float32)
         + bl_a[...] + bl_b[...]) * 0.5
    h = jnp.where(h >= 0, h, 0.01 * h)
    out[...] = (jnp.dot(h, lin_w[...], preferred_element_type=jnp.float32)
                + lin_b[...])


def _tc_epilogue(acc_a, cnt_a, acc_b, cnt_b, x_product,
                 wl_a, bl_a, wr_a, wl_b, bl_b, wr_b, lin_w, lin_b):
    n = x_product.shape[0]
    n_cls = lin_w.shape[1]
    bm = 1000
    grid = (n // bm,)
    row_spec = lambda w: pl.BlockSpec((bm, w), lambda i: (i, 0))
    full = lambda a: pl.BlockSpec(a.shape, lambda i: (0,) * a.ndim)
    return pl.pallas_call(
        _tc_body,
        grid=grid,
        in_specs=[
            row_spec(D_IN), row_spec(1), row_spec(D_IN), row_spec(1),
            row_spec(D_IN),
            full(wl_a), full(bl_a), full(wr_a),
            full(wl_b), full(bl_b), full(wr_b),
            full(lin_w), full(lin_b),
        ],
        out_specs=row_spec(n_cls),
        out_shape=jax.ShapeDtypeStruct((n, n_cls), jnp.float32),
    )(acc_a, cnt_a, acc_b, cnt_b, x_product,
      wl_a, bl_a, wr_a, wl_b, bl_b, wr_b, lin_w, lin_b)


def kernel(x_product, x_demographic, x_platform, edge_index_targets,
           edge_index_rev_targets, edge_index_uses, edge_index_rev_uses,
           edge_index_self,
           Wl_targets, bl_targets, Wr_targets,
           Wl_rev_targets, bl_rev_targets, Wr_rev_targets,
           Wl_uses, bl_uses, Wr_uses,
           Wl_rev_uses, bl_rev_uses, Wr_rev_uses,
           Wl_self, bl_self, Wr_self,
           lin_W, lin_b):
    n_prod = x_product.shape[0]
    n_demo = x_demographic.shape[0]

    x_all = jnp.concatenate([x_demographic, x_product], axis=0)
    edges_all = jnp.stack([
        _prep_edges(edge_index_rev_targets, 0, n_prod),
        _prep_edges(edge_index_self, n_demo, n_prod),
    ], axis=0)

    acc_a, cnt_a, acc_b, cnt_b = _sc_accumulate(n_prod, edges_all, x_all)

    return _tc_epilogue(
        acc_a, cnt_a.reshape(n_prod, 1), acc_b, cnt_b.reshape(n_prod, 1),
        x_product,
        Wl_rev_targets, bl_rev_targets.reshape(1, -1), Wr_rev_targets,
        Wl_self, bl_self.reshape(1, -1), Wr_self,
        lin_W, lin_b.reshape(1, -1))
